# baseline (speedup 1.0000x reference)
import functools

import jax
import jax.numpy as jnp
from jax import lax
from jax.experimental import pallas as pl
from jax.experimental.pallas import tpu as pltpu

N = 10000
D = 128
BM = 2000

NC = 2
NS = 16
NT = NC * NS
CB = 80
E_PAD = 327680
EPT = E_PAD // NT
NCH = EPT // CB
RPT = 632
NR = NS * RPT
RD = 4

_sc_mesh = None


def _get_sc_mesh():
    global _sc_mesh
    if _sc_mesh is None:
        from jax.experimental.pallas import tpu_sc as plsc
        _sc_mesh = plsc.VectorSubcoreMesh(
            core_axis_name="c", subcore_axis_name="s",
            num_cores=NC, num_subcores=NS)
    return _sc_mesh


def _mm_body(a_ref, w_ref, b_ref, o_ref):
    y = jnp.dot(a_ref[...], w_ref[...], preferred_element_type=jnp.float32)
    o_ref[...] = y + b_ref[...]


def _mm(a, w, b):
    return pl.pallas_call(
        _mm_body,
        grid=(N // BM,),
        in_specs=[
            pl.BlockSpec((BM, D), lambda i: (i, 0)),
            pl.BlockSpec((D, D), lambda i: (0, 0)),
            pl.BlockSpec((1, D), lambda i: (0, 0)),
        ],
        out_specs=pl.BlockSpec((BM, D), lambda i: (i, 0)),
        out_shape=jax.ShapeDtypeStruct((N, D), jnp.float32),
    )(a, w, b.reshape(1, D))


def _masked_mm_body(x_ref, m_ref, w_ref, b_ref, o_ref):
    xm = x_ref[...] * m_ref[...]
    o_ref[...] = jnp.dot(xm, w_ref[...], preferred_element_type=jnp.float32) + b_ref[...]


def _masked_mm(x, mask, w, b):
    return pl.pallas_call(
        _masked_mm_body,
        grid=(N // BM,),
        in_specs=[
            pl.BlockSpec((BM, D), lambda i: (i, 0)),
            pl.BlockSpec((BM, D), lambda i: (i, 0)),
            pl.BlockSpec((D, D), lambda i: (0, 0)),
            pl.BlockSpec((1, D), lambda i: (0, 0)),
        ],
        out_specs=pl.BlockSpec((BM, D), lambda i: (i, 0)),
        out_shape=jax.ShapeDtypeStruct((N, D), jnp.float32),
    )(x, mask, w, b.reshape(1, D))


def _mask_mlp_body(agA_ref, agB_ref, xl2_ref, m1t_ref, m1b_ref, b1_ref,
                   m2_ref, b2_ref, o_ref):
    a = jnp.maximum(agA_ref[...] + agB_ref[...], 0.0)
    cx = jnp.maximum(xl2_ref[...], 0.0)
    w = (jnp.dot(a, m1t_ref[...], preferred_element_type=jnp.float32)
         + jnp.dot(cx, m1b_ref[...], preferred_element_type=jnp.float32)
         + b1_ref[...])
    w = jnp.maximum(w, 0.0)
    y = jnp.dot(w, m2_ref[...], preferred_element_type=jnp.float32) + b2_ref[...]
    o_ref[...] = jax.nn.sigmoid(y)


def _mask_mlp(agA, agB, xl2, m1W, m1b, m2W, m2b):
    return pl.pallas_call(
        _mask_mlp_body,
        grid=(N // BM,),
        in_specs=[
            pl.BlockSpec((BM, D), lambda i: (i, 0)),
            pl.BlockSpec((BM, D), lambda i: (i, 0)),
            pl.BlockSpec((BM, D), lambda i: (i, 0)),
            pl.BlockSpec((D, D), lambda i: (0, 0)),
            pl.BlockSpec((D, D), lambda i: (0, 0)),
            pl.BlockSpec((1, D), lambda i: (0, 0)),
            pl.BlockSpec((D, D), lambda i: (0, 0)),
            pl.BlockSpec((1, D), lambda i: (0, 0)),
        ],
        out_specs=pl.BlockSpec((BM, D), lambda i: (i, 0)),
        out_shape=jax.ShapeDtypeStruct((N, D), jnp.float32),
    )(agA, agB, xl2, m1W[:D], m1W[D:], m1b.reshape(1, D), m2W,
      m2b.reshape(1, D))


def _combine_body(agA_ref, agB_ref, x_ref, lw_ref, m_ref, o_ref):
    y = (agA_ref[...] + agB_ref[...]
         + jnp.dot(x_ref[...], lw_ref[...], preferred_element_type=jnp.float32))
    o_ref[...] = jnp.maximum(y * m_ref[...], 0.0)


def _combine(agA, agB, x, linW, mask):
    return pl.pallas_call(
        _combine_body,
        grid=(N // BM,),
        in_specs=[
            pl.BlockSpec((BM, D), lambda i: (i, 0)),
            pl.BlockSpec((BM, D), lambda i: (i, 0)),
            pl.BlockSpec((BM, D), lambda i: (i, 0)),
            pl.BlockSpec((D, D), lambda i: (0, 0)),
            pl.BlockSpec((BM, D), lambda i: (i, 0)),
        ],
        out_specs=pl.BlockSpec((BM, D), lambda i: (i, 0)),
        out_shape=jax.ShapeDtypeStruct((N, D), jnp.float32),
    )(agA, agB, x, linW, mask)


def _post_body(x_ref, p1_ref, b1_ref, p2_ref, b2_ref, o_ref):
    y = jnp.dot(x_ref[...], p1_ref[...], preferred_element_type=jnp.float32) + b1_ref[...]
    y = jnp.maximum(y, 0.0)
    o_ref[...] = jnp.dot(y, p2_ref[...], preferred_element_type=jnp.float32) + b2_ref[...]


def _post(x, p1W, p1b, p2W, p2b):
    return pl.pallas_call(
        _post_body,
        grid=(N // BM,),
        in_specs=[
            pl.BlockSpec((BM, D), lambda i: (i, 0)),
            pl.BlockSpec((D, D), lambda i: (0, 0)),
            pl.BlockSpec((1, D), lambda i: (0, 0)),
            pl.BlockSpec((D, D), lambda i: (0, 0)),
            pl.BlockSpec((1, D), lambda i: (0, 0)),
        ],
        out_specs=pl.BlockSpec((BM, D), lambda i: (i, 0)),
        out_shape=jax.ShapeDtypeStruct((N, D), jnp.float32),
    )(x, p1W, p1b.reshape(1, D), p2W, p2b.reshape(1, D))


def _segsum_sc_body(h_hbm, src_hbm, dst_hbm, ew_hbm, out_hbm,
                    sring, dring, ering, r0, r1, r2, r3, acc_sh,
                    isS0, isS1, isS2, isS3, isD0, isD1, isD2, isD3,
                    isE0, isE1, isE2, isE3, g0, g1, g2, g3, s0, s1, s2, s3):
    from jax.experimental.pallas import tpu_sc as plsc
    c = lax.axis_index("c")
    s = lax.axis_index("s")
    wid = s * NC + c
    ch0 = wid * NCH

    isS = (isS0, isS1, isS2, isS3)
    isD = (isD0, isD1, isD2, isD3)
    isE = (isE0, isE1, isE2, isE3)
    gse = (g0, g1, g2, g3)
    sse = (s0, s1, s2, s3)
    bufs = (r0, r1, r2, r3)

    def issue_idxS(j, slot):
        pltpu.async_copy(src_hbm.at[ch0 + j], sring.at[slot], isS[slot])

    def wait_idxS(j, slot):
        pltpu.make_async_copy(src_hbm.at[ch0 + j], sring.at[slot],
                              isS[slot]).wait()

    def issue_idxD(j, slot):
        pltpu.async_copy(dst_hbm.at[ch0 + j], dring.at[slot], isD[slot])
        pltpu.async_copy(ew_hbm.at[ch0 + j], ering.at[slot], isE[slot])

    def wait_idxD(j, slot):
        pltpu.make_async_copy(dst_hbm.at[ch0 + j], dring.at[slot],
                              isD[slot]).wait()

    def wait_ew(j, slot):
        pltpu.make_async_copy(ew_hbm.at[ch0 + j], ering.at[slot],
                              isE[slot]).wait()

    def issue_gather(j, slot):
        pltpu.async_copy(h_hbm.at[sring.at[slot]], bufs[slot], gse[slot])

    def wait_gather(j, slot):
        pltpu.make_async_copy(h_hbm.at[sring.at[slot]], bufs[slot],
                              gse[slot]).wait()

    def issue_scatter(j, slot):
        pltpu.async_copy(bufs[slot], acc_sh.at[dring.at[slot]], sse[slot],
                         add=True)

    def wait_scatter(j, slot):
        pltpu.make_async_copy(bufs[slot], acc_sh.at[dring.at[slot]],
                              sse[slot]).wait()

    def scale(slot, ch):
        def sgroup(g, _):
            ewg = ering[slot, pl.ds(g * 16, 16)]
            for r2 in range(16):
                m = jnp.broadcast_to(ewg[r2], (16,))
                for f in range(D // 16):
                    sl = pl.ds(f * 16, 16)
                    bufs[slot][g * 16 + r2, sl] = bufs[slot][g * 16 + r2, sl] * m
            return 0

        lax.fori_loop(0, CB // 16, sgroup, 0)

    for k in range(3):
        issue_idxS(k, k)
    issue_idxD(0, 0)
    issue_idxD(1, 1)
    wait_idxS(0, 0)
    issue_gather(0, 0)
    wait_idxS(1, 1)
    issue_gather(1, 1)

    z = jnp.zeros((16,), jnp.float32)

    def zrow(r, _):
        for f in range(D // 16):
            r3[r, pl.ds(f * 16, 16)] = z
        return 0

    lax.fori_loop(0, CB, zrow, 0, unroll=4)
    row0 = s * RPT
    for off in range(0, RPT, CB):
        nr = min(CB, RPT - off)
        pltpu.sync_copy(r3.at[pl.ds(0, nr)], acc_sh.at[pl.ds(row0 + off, nr)])
    plsc.subcore_barrier()

    def step(j, k, *, wS, wG2, wI3):
        if wG2:
            wait_idxS(j + 2, (k + 2) % RD)
        if wS:
            wait_scatter(j - 2, (k + 2) % RD)
        if wG2:
            issue_gather(j + 2, (k + 2) % RD)
            issue_idxD(j + 2, (k + 2) % RD)
        if wI3:
            issue_idxS(j + 3, (k + 3) % RD)
        wait_gather(j, k)
        wait_ew(j, k)
        scale(k, j)
        wait_idxD(j, k)
        issue_scatter(j, k)

    step(0, 0, wS=False, wG2=True, wI3=True)
    step(1, 1, wS=False, wG2=True, wI3=True)
    step(2, 2, wS=True, wG2=True, wI3=True)
    step(3, 3, wS=True, wG2=True, wI3=True)

    def body(i, _):
        j0 = i * 4
        for k in range(4):
            step(j0 + k, k, wS=True, wG2=True, wI3=True)
        return 0

    lax.fori_loop(1, (NCH - 4) // 4, body, 0)

    step(NCH - 4, 0, wS=True, wG2=True, wI3=True)
    step(NCH - 3, 1, wS=True, wG2=True, wI3=False)
    step(NCH - 2, 2, wS=True, wG2=False, wI3=False)
    step(NCH - 1, 3, wS=True, wG2=False, wI3=False)
    wait_scatter(NCH - 2, 2)
    wait_scatter(NCH - 1, 3)

    plsc.subcore_barrier()
    for off in range(0, RPT, CB):
        nr = min(CB, RPT - off)
        pltpu.sync_copy(acc_sh.at[pl.ds(row0 + off, nr)],
                        out_hbm.at[c, pl.ds(row0 + off, nr)])


def _segsum_partials(h, srcc, dstc, ewc):
    f = pl.kernel(
        _segsum_sc_body,
        out_type=jax.ShapeDtypeStruct((NC, NR, D), jnp.float32),
        mesh=_get_sc_mesh(),
        scratch_types=(
            [pltpu.VMEM((RD, CB), jnp.int32),
             pltpu.VMEM((RD, CB), jnp.int32),
             pltpu.VMEM((RD, CB), jnp.float32)]
            + [pltpu.VMEM((CB, D), jnp.float32)] * 4
            + [pltpu.VMEM_SHARED((NR, D), jnp.float32)]
            + [pltpu.SemaphoreType.DMA] * 20
        ),
    )
    return f(h, srcc, dstc, ewc)


def _segsum(h, srcc, dstc, ewc):
    out = _segsum_partials(h, srcc, dstc, ewc)
    return out[0, :N], out[1, :N]


def kernel(x, edge_attr, edge_index, W0, lin0W, W1, lin1W,
           m0_l1W, m0_l1b, m0_l2W, m0_l2b, m0_m1W, m0_m1b, m0_m2W, m0_m2b,
           m1_l1W, m1_l1b, m1_l2W, m1_l2b, m1_m1W, m1_m1b, m1_m2W, m1_m2b,
           p1W, p1b, p2W, p2b):
    pad = E_PAD - edge_attr.shape[0]
    srcc = jnp.pad(edge_index[0], (0, pad)).reshape(E_PAD // CB, CB)
    dstc = jnp.pad(edge_index[1], (0, pad)).reshape(E_PAD // CB, CB)
    ew = jnp.pad(edge_attr, (0, pad)).reshape(E_PAD // CB, CB)

    zb = jnp.zeros((D,), jnp.float32)

    h1 = _mm(x, m0_l1W, m0_l1b)
    agA, agB = _segsum(h1, srcc, dstc, ew)
    xl2 = _mm(x, m0_l2W, m0_l2b)
    mask0 = _mask_mlp(agA, agB, xl2, m0_m1W, m0_m1b, m0_m2W, m0_m2b)

    h2 = _masked_mm(x, mask0, W0, zb)
    agA, agB = _segsum(h2, srcc, dstc, ew)
    x1 = _combine(agA, agB, x, lin0W, mask0)

    h1 = _masked_mm(x1, mask0, m1_l1W, m1_l1b)
    agA, agB = _segsum(h1, srcc, dstc, ew)
    xl2 = _masked_mm(x1, mask0, m1_l2W, m1_l2b)
    mask1 = _mask_mlp(agA, agB, xl2, m1_m1W, m1_m1b, m1_m2W, m1_m2b)

    h2 = _masked_mm(x1, mask1, W1, zb)
    agA, agB = _segsum(h2, srcc, dstc, ew)
    x2 = _combine(agA, agB, x1, lin1W, mask1)

    return _post(x2, p1W, p1b, p2W, p2b)


# speedup vs baseline: 3.5866x; 3.5866x over previous
import functools

import jax
import jax.numpy as jnp
from jax import lax
from jax.experimental import pallas as pl
from jax.experimental.pallas import tpu as pltpu

N = 10000
D = 128
BM = 2000

NC = 2
NS = 16
NT = NC * NS
CB = 80
E_PAD = 327680
EPT = E_PAD // NT
NCH = EPT // CB
RPT = 632
NR = NS * RPT
RD = 4

_sc_mesh = None


def _get_sc_mesh():
    global _sc_mesh
    if _sc_mesh is None:
        from jax.experimental.pallas import tpu_sc as plsc
        _sc_mesh = plsc.VectorSubcoreMesh(
            core_axis_name="c", subcore_axis_name="s",
            num_cores=NC, num_subcores=NS)
    return _sc_mesh


def _mm_body(a_ref, w_ref, b_ref, o_ref):
    y = jnp.dot(a_ref[...], w_ref[...], preferred_element_type=jnp.float32)
    o_ref[...] = y + b_ref[...]


def _mm(a, w, b):
    return pl.pallas_call(
        _mm_body,
        grid=(N // BM,),
        in_specs=[
            pl.BlockSpec((BM, D), lambda i: (i, 0)),
            pl.BlockSpec((D, D), lambda i: (0, 0)),
            pl.BlockSpec((1, D), lambda i: (0, 0)),
        ],
        out_specs=pl.BlockSpec((BM, D), lambda i: (i, 0)),
        out_shape=jax.ShapeDtypeStruct((N, D), jnp.float32),
    )(a, w, b.reshape(1, D))


def _masked_mm_body(x_ref, m_ref, w_ref, b_ref, o_ref):
    xm = x_ref[...] * m_ref[...]
    o_ref[...] = jnp.dot(xm, w_ref[...], preferred_element_type=jnp.float32) + b_ref[...]


def _masked_mm(x, mask, w, b):
    return pl.pallas_call(
        _masked_mm_body,
        grid=(N // BM,),
        in_specs=[
            pl.BlockSpec((BM, D), lambda i: (i, 0)),
            pl.BlockSpec((BM, D), lambda i: (i, 0)),
            pl.BlockSpec((D, D), lambda i: (0, 0)),
            pl.BlockSpec((1, D), lambda i: (0, 0)),
        ],
        out_specs=pl.BlockSpec((BM, D), lambda i: (i, 0)),
        out_shape=jax.ShapeDtypeStruct((N, D), jnp.float32),
    )(x, mask, w, b.reshape(1, D))


def _mask_mlp_body(agA_ref, agB_ref, xl2_ref, m1t_ref, m1b_ref, b1_ref,
                   m2_ref, b2_ref, o_ref):
    a = jnp.maximum(agA_ref[...] + agB_ref[...], 0.0)
    cx = jnp.maximum(xl2_ref[...], 0.0)
    w = (jnp.dot(a, m1t_ref[...], preferred_element_type=jnp.float32)
         + jnp.dot(cx, m1b_ref[...], preferred_element_type=jnp.float32)
         + b1_ref[...])
    w = jnp.maximum(w, 0.0)
    y = jnp.dot(w, m2_ref[...], preferred_element_type=jnp.float32) + b2_ref[...]
    o_ref[...] = jax.nn.sigmoid(y)


def _mask_mlp(agA, agB, xl2, m1W, m1b, m2W, m2b):
    return pl.pallas_call(
        _mask_mlp_body,
        grid=(N // BM,),
        in_specs=[
            pl.BlockSpec((BM, D), lambda i: (i, 0)),
            pl.BlockSpec((BM, D), lambda i: (i, 0)),
            pl.BlockSpec((BM, D), lambda i: (i, 0)),
            pl.BlockSpec((D, D), lambda i: (0, 0)),
            pl.BlockSpec((D, D), lambda i: (0, 0)),
            pl.BlockSpec((1, D), lambda i: (0, 0)),
            pl.BlockSpec((D, D), lambda i: (0, 0)),
            pl.BlockSpec((1, D), lambda i: (0, 0)),
        ],
        out_specs=pl.BlockSpec((BM, D), lambda i: (i, 0)),
        out_shape=jax.ShapeDtypeStruct((N, D), jnp.float32),
    )(agA, agB, xl2, m1W[:D], m1W[D:], m1b.reshape(1, D), m2W,
      m2b.reshape(1, D))


def _combine_body(agA_ref, agB_ref, x_ref, lw_ref, m_ref, o_ref):
    y = (agA_ref[...] + agB_ref[...]
         + jnp.dot(x_ref[...], lw_ref[...], preferred_element_type=jnp.float32))
    o_ref[...] = jnp.maximum(y * m_ref[...], 0.0)


def _combine(agA, agB, x, linW, mask):
    return pl.pallas_call(
        _combine_body,
        grid=(N // BM,),
        in_specs=[
            pl.BlockSpec((BM, D), lambda i: (i, 0)),
            pl.BlockSpec((BM, D), lambda i: (i, 0)),
            pl.BlockSpec((BM, D), lambda i: (i, 0)),
            pl.BlockSpec((D, D), lambda i: (0, 0)),
            pl.BlockSpec((BM, D), lambda i: (i, 0)),
        ],
        out_specs=pl.BlockSpec((BM, D), lambda i: (i, 0)),
        out_shape=jax.ShapeDtypeStruct((N, D), jnp.float32),
    )(agA, agB, x, linW, mask)


def _post_body(x_ref, p1_ref, b1_ref, p2_ref, b2_ref, o_ref):
    y = jnp.dot(x_ref[...], p1_ref[...], preferred_element_type=jnp.float32) + b1_ref[...]
    y = jnp.maximum(y, 0.0)
    o_ref[...] = jnp.dot(y, p2_ref[...], preferred_element_type=jnp.float32) + b2_ref[...]


def _post(x, p1W, p1b, p2W, p2b):
    return pl.pallas_call(
        _post_body,
        grid=(N // BM,),
        in_specs=[
            pl.BlockSpec((BM, D), lambda i: (i, 0)),
            pl.BlockSpec((D, D), lambda i: (0, 0)),
            pl.BlockSpec((1, D), lambda i: (0, 0)),
            pl.BlockSpec((D, D), lambda i: (0, 0)),
            pl.BlockSpec((1, D), lambda i: (0, 0)),
        ],
        out_specs=pl.BlockSpec((BM, D), lambda i: (i, 0)),
        out_shape=jax.ShapeDtypeStruct((N, D), jnp.float32),
    )(x, p1W, p1b.reshape(1, D), p2W, p2b.reshape(1, D))


def _segsum_sc_body(h_hbm, src_hbm, dst_hbm, ew_hbm, out_hbm,
                    sring, dring, ering, r0, r1, r2, r3, acc_sh,
                    isS0, isS1, isS2, isS3, isD0, isD1, isD2, isD3,
                    isE0, isE1, isE2, isE3, g0, g1, g2, g3, s0, s1, s2, s3):
    from jax.experimental.pallas import tpu_sc as plsc
    c = lax.axis_index("c")
    s = lax.axis_index("s")
    wid = s * NC + c
    ch0 = wid * NCH

    isS = (isS0, isS1, isS2, isS3)
    isD = (isD0, isD1, isD2, isD3)
    isE = (isE0, isE1, isE2, isE3)
    gse = (g0, g1, g2, g3)
    sse = (s0, s1, s2, s3)
    bufs = (r0, r1, r2, r3)

    def issue_idxS(j, slot):
        pltpu.async_copy(src_hbm.at[ch0 + j], sring.at[slot], isS[slot])

    def wait_idxS(j, slot):
        pltpu.make_async_copy(src_hbm.at[ch0 + j], sring.at[slot],
                              isS[slot]).wait()

    def issue_idxD(j, slot):
        pltpu.async_copy(dst_hbm.at[ch0 + j], dring.at[slot], isD[slot])
        pltpu.async_copy(ew_hbm.at[ch0 + j], ering.at[slot], isE[slot])

    def wait_idxD(j, slot):
        pltpu.make_async_copy(dst_hbm.at[ch0 + j], dring.at[slot],
                              isD[slot]).wait()

    def wait_ew(j, slot):
        pltpu.make_async_copy(ew_hbm.at[ch0 + j], ering.at[slot],
                              isE[slot]).wait()

    def issue_gather(j, slot):
        pltpu.async_copy(h_hbm.at[sring.at[slot]], bufs[slot], gse[slot])

    def wait_gather(j, slot):
        pltpu.make_async_copy(h_hbm.at[sring.at[slot]], bufs[slot],
                              gse[slot]).wait()

    def issue_scatter(j, slot):
        pltpu.async_copy(bufs[slot], acc_sh.at[dring.at[slot]], sse[slot],
                         add=True)

    def wait_scatter(j, slot):
        pltpu.make_async_copy(bufs[slot], acc_sh.at[dring.at[slot]],
                              sse[slot]).wait()

    def scale(slot, ch):
        def sgroup(g, _):
            ewg = ering[slot, pl.ds(g * 16, 16)]
            for r2 in range(16):
                m = jnp.broadcast_to(ewg[r2], (16,))
                for f in range(D // 16):
                    sl = pl.ds(f * 16, 16)
                    bufs[slot][g * 16 + r2, sl] = bufs[slot][g * 16 + r2, sl] * m
            return 0

        lax.fori_loop(0, CB // 16, sgroup, 0)

    for k in range(3):
        issue_idxS(k, k)
    issue_idxD(0, 0)
    issue_idxD(1, 1)
    wait_idxS(0, 0)
    issue_gather(0, 0)
    wait_idxS(1, 1)
    issue_gather(1, 1)

    z = jnp.zeros((16,), jnp.float32)

    def zrow(r, _):
        for f in range(D // 16):
            r3[r, pl.ds(f * 16, 16)] = z
        return 0

    lax.fori_loop(0, CB, zrow, 0, unroll=4)
    row0 = s * RPT
    for off in range(0, RPT, CB):
        nr = min(CB, RPT - off)
        pltpu.sync_copy(r3.at[pl.ds(0, nr)], acc_sh.at[pl.ds(row0 + off, nr)])
    plsc.subcore_barrier()

    def step(j, k, *, wS, wG2, wI3):
        if wG2:
            wait_idxS(j + 2, (k + 2) % RD)
        if wS:
            wait_scatter(j - 2, (k + 2) % RD)
        if wG2:
            issue_gather(j + 2, (k + 2) % RD)
            issue_idxD(j + 2, (k + 2) % RD)
        if wI3:
            issue_idxS(j + 3, (k + 3) % RD)
        wait_gather(j, k)
        wait_ew(j, k)
        scale(k, j)
        wait_idxD(j, k)
        issue_scatter(j, k)

    step(0, 0, wS=False, wG2=True, wI3=True)
    step(1, 1, wS=False, wG2=True, wI3=True)
    step(2, 2, wS=True, wG2=True, wI3=True)
    step(3, 3, wS=True, wG2=True, wI3=True)

    def body(i, _):
        j0 = i * 4
        for k in range(4):
            step(j0 + k, k, wS=True, wG2=True, wI3=True)
        return 0

    lax.fori_loop(1, (NCH - 4) // 4, body, 0)

    step(NCH - 4, 0, wS=True, wG2=True, wI3=True)
    step(NCH - 3, 1, wS=True, wG2=True, wI3=False)
    step(NCH - 2, 2, wS=True, wG2=False, wI3=False)
    step(NCH - 1, 3, wS=True, wG2=False, wI3=False)
    wait_scatter(NCH - 2, 2)
    wait_scatter(NCH - 1, 3)

    plsc.subcore_barrier()
    for off in range(0, RPT, CB):
        nr = min(CB, RPT - off)
        pltpu.sync_copy(acc_sh.at[pl.ds(row0 + off, nr)],
                        out_hbm.at[c, pl.ds(row0 + off, nr)])


def _segsum_partials(h, srcc, dstc, ewc):
    f = pl.kernel(
        _segsum_sc_body,
        out_type=jax.ShapeDtypeStruct((NC, NR, D), jnp.float32),
        mesh=_get_sc_mesh(),
        scratch_types=(
            [pltpu.VMEM((RD, CB), jnp.int32),
             pltpu.VMEM((RD, CB), jnp.int32),
             pltpu.VMEM((RD, CB), jnp.float32)]
            + [pltpu.VMEM((CB, D), jnp.float32)] * 4
            + [pltpu.VMEM_SHARED((NR, D), jnp.float32)]
            + [pltpu.SemaphoreType.DMA] * 20
        ),
    )
    return f(h, srcc, dstc, ewc)


def _segsum(h, srcc, dstc, ewc):
    out = _segsum_partials(h, srcc, dstc, ewc)
    return out[0, :N], out[1, :N]


def kernel(x, edge_attr, edge_index, W0, lin0W, W1, lin1W,
           m0_l1W, m0_l1b, m0_l2W, m0_l2b, m0_m1W, m0_m1b, m0_m2W, m0_m2b,
           m1_l1W, m1_l1b, m1_l2W, m1_l2b, m1_m1W, m1_m1b, m1_m2W, m1_m2b,
           p1W, p1b, p2W, p2b):
    pad = E_PAD - edge_attr.shape[0]
    spread = (jnp.arange(pad, dtype=edge_index.dtype) * 13) % N
    srcc = jnp.concatenate([edge_index[0], spread]).reshape(E_PAD // CB, CB)
    dstc = jnp.concatenate([edge_index[1], spread]).reshape(E_PAD // CB, CB)
    ew = jnp.pad(edge_attr, (0, pad)).reshape(E_PAD // CB, CB)

    zb = jnp.zeros((D,), jnp.float32)

    h1 = _mm(x, m0_l1W, m0_l1b)
    agA, agB = _segsum(h1, srcc, dstc, ew)
    xl2 = _mm(x, m0_l2W, m0_l2b)
    mask0 = _mask_mlp(agA, agB, xl2, m0_m1W, m0_m1b, m0_m2W, m0_m2b)

    h2 = _masked_mm(x, mask0, W0, zb)
    agA, agB = _segsum(h2, srcc, dstc, ew)
    x1 = _combine(agA, agB, x, lin0W, mask0)

    h1 = _masked_mm(x1, mask0, m1_l1W, m1_l1b)
    agA, agB = _segsum(h1, srcc, dstc, ew)
    xl2 = _masked_mm(x1, mask0, m1_l2W, m1_l2b)
    mask1 = _mask_mlp(agA, agB, xl2, m1_m1W, m1_m1b, m1_m2W, m1_m2b)

    h2 = _masked_mm(x1, mask1, W1, zb)
    agA, agB = _segsum(h2, srcc, dstc, ew)
    x2 = _combine(agA, agB, x1, lin1W, mask1)

    return _post(x2, p1W, p1b, p2W, p2b)


# speedup vs baseline: 3.9839x; 1.1108x over previous
import functools

import jax
import jax.numpy as jnp
from jax import lax
from jax.experimental import pallas as pl
from jax.experimental.pallas import tpu as pltpu

N = 10000
D = 128
BM = 2000

NC = 2
NS = 16
NT = NC * NS
CB = 80
E = 320000
EPT = E // NT
NCH = EPT // CB
RPT = 632
NR = NS * RPT
RD = 4

_sc_mesh = None


def _get_sc_mesh():
    global _sc_mesh
    if _sc_mesh is None:
        from jax.experimental.pallas import tpu_sc as plsc
        _sc_mesh = plsc.VectorSubcoreMesh(
            core_axis_name="c", subcore_axis_name="s",
            num_cores=NC, num_subcores=NS)
    return _sc_mesh


def _mm_body(a_ref, w_ref, b_ref, o_ref):
    y = jnp.dot(a_ref[...], w_ref[...], preferred_element_type=jnp.float32)
    o_ref[...] = y + b_ref[...]


def _mm(a, w, b):
    return pl.pallas_call(
        _mm_body,
        grid=(N // BM,),
        in_specs=[
            pl.BlockSpec((BM, D), lambda i: (i, 0)),
            pl.BlockSpec((D, D), lambda i: (0, 0)),
            pl.BlockSpec((1, D), lambda i: (0, 0)),
        ],
        out_specs=pl.BlockSpec((BM, D), lambda i: (i, 0)),
        out_shape=jax.ShapeDtypeStruct((N, D), jnp.float32),
    )(a, w, b.reshape(1, D))


def _ag_specs():
    return [
        pl.BlockSpec((1, BM, D), lambda i: (0, i, 0)),
        pl.BlockSpec((1, BM, D), lambda i: (1, i, 0)),
    ]


def _mask_h2_body(agA_ref, agB_ref, xl2_ref, x_ref, m1t_ref, m1b_ref,
                  b1_ref, m2_ref, b2_ref, w_ref, mask_ref, h2_ref):
    a = jnp.maximum(agA_ref[0] + agB_ref[0], 0.0)
    cx = jnp.maximum(xl2_ref[...], 0.0)
    w = (jnp.dot(a, m1t_ref[...], preferred_element_type=jnp.float32)
         + jnp.dot(cx, m1b_ref[...], preferred_element_type=jnp.float32)
         + b1_ref[...])
    w = jnp.maximum(w, 0.0)
    y = jnp.dot(w, m2_ref[...], preferred_element_type=jnp.float32) + b2_ref[...]
    mask = jax.nn.sigmoid(y)
    mask_ref[...] = mask
    xm = x_ref[...] * mask
    h2_ref[...] = jnp.dot(xm, w_ref[...], preferred_element_type=jnp.float32)


def _mask_h2(ag, xl2, x, m1W, m1b, m2W, m2b, W):
    return pl.pallas_call(
        _mask_h2_body,
        grid=(N // BM,),
        in_specs=_ag_specs() + [
            pl.BlockSpec((BM, D), lambda i: (i, 0)),
            pl.BlockSpec((BM, D), lambda i: (i, 0)),
            pl.BlockSpec((D, D), lambda i: (0, 0)),
            pl.BlockSpec((D, D), lambda i: (0, 0)),
            pl.BlockSpec((1, D), lambda i: (0, 0)),
            pl.BlockSpec((D, D), lambda i: (0, 0)),
            pl.BlockSpec((1, D), lambda i: (0, 0)),
            pl.BlockSpec((D, D), lambda i: (0, 0)),
        ],
        out_specs=[pl.BlockSpec((BM, D), lambda i: (i, 0)),
                   pl.BlockSpec((BM, D), lambda i: (i, 0))],
        out_shape=[jax.ShapeDtypeStruct((N, D), jnp.float32),
                   jax.ShapeDtypeStruct((N, D), jnp.float32)],
    )(ag, ag, xl2, x, m1W[:D], m1W[D:], m1b.reshape(1, D), m2W,
      m2b.reshape(1, D), W)


def _comb_h1l2_body(agA_ref, agB_ref, x_ref, lw_ref, m_ref, l1W_ref, l1b_ref,
                    l2W_ref, l2b_ref, x1_ref, h1_ref, xl2_ref):
    y = (agA_ref[0] + agB_ref[0]
         + jnp.dot(x_ref[...], lw_ref[...], preferred_element_type=jnp.float32))
    x1 = jnp.maximum(y * m_ref[...], 0.0)
    x1_ref[...] = x1
    xm = x1 * m_ref[...]
    h1_ref[...] = jnp.dot(xm, l1W_ref[...], preferred_element_type=jnp.float32) + l1b_ref[...]
    xl2_ref[...] = jnp.dot(xm, l2W_ref[...], preferred_element_type=jnp.float32) + l2b_ref[...]


def _comb_h1l2(ag, x, linW, mask, l1W, l1b, l2W, l2b):
    return pl.pallas_call(
        _comb_h1l2_body,
        grid=(N // BM,),
        in_specs=_ag_specs() + [
            pl.BlockSpec((BM, D), lambda i: (i, 0)),
            pl.BlockSpec((D, D), lambda i: (0, 0)),
            pl.BlockSpec((BM, D), lambda i: (i, 0)),
            pl.BlockSpec((D, D), lambda i: (0, 0)),
            pl.BlockSpec((1, D), lambda i: (0, 0)),
            pl.BlockSpec((D, D), lambda i: (0, 0)),
            pl.BlockSpec((1, D), lambda i: (0, 0)),
        ],
        out_specs=[pl.BlockSpec((BM, D), lambda i: (i, 0))] * 3,
        out_shape=[jax.ShapeDtypeStruct((N, D), jnp.float32)] * 3,
    )(ag, ag, x, linW, mask, l1W, l1b.reshape(1, D), l2W, l2b.reshape(1, D))


def _comb_post_body(agA_ref, agB_ref, x_ref, lw_ref, m_ref, p1_ref, b1_ref,
                    p2_ref, b2_ref, o_ref):
    y = (agA_ref[0] + agB_ref[0]
         + jnp.dot(x_ref[...], lw_ref[...], preferred_element_type=jnp.float32))
    x2 = jnp.maximum(y * m_ref[...], 0.0)
    y = jnp.dot(x2, p1_ref[...], preferred_element_type=jnp.float32) + b1_ref[...]
    y = jnp.maximum(y, 0.0)
    o_ref[...] = jnp.dot(y, p2_ref[...], preferred_element_type=jnp.float32) + b2_ref[...]


def _comb_post(ag, x, linW, mask, p1W, p1b, p2W, p2b):
    return pl.pallas_call(
        _comb_post_body,
        grid=(N // BM,),
        in_specs=_ag_specs() + [
            pl.BlockSpec((BM, D), lambda i: (i, 0)),
            pl.BlockSpec((D, D), lambda i: (0, 0)),
            pl.BlockSpec((BM, D), lambda i: (i, 0)),
            pl.BlockSpec((D, D), lambda i: (0, 0)),
            pl.BlockSpec((1, D), lambda i: (0, 0)),
            pl.BlockSpec((D, D), lambda i: (0, 0)),
            pl.BlockSpec((1, D), lambda i: (0, 0)),
        ],
        out_specs=pl.BlockSpec((BM, D), lambda i: (i, 0)),
        out_shape=jax.ShapeDtypeStruct((N, D), jnp.float32),
    )(ag, ag, x, linW, mask, p1W, p1b.reshape(1, D), p2W, p2b.reshape(1, D))


def _segsum_sc_body(h_hbm, src_hbm, dst_hbm, ew_hbm, out_hbm,
                    sring, dring, ering, r0, r1, r2, r3, acc_sh,
                    isS0, isS1, isS2, isS3, isD0, isD1, isD2, isD3,
                    isE0, isE1, isE2, isE3, g0, g1, g2, g3, s0, s1, s2, s3):
    from jax.experimental.pallas import tpu_sc as plsc
    c = lax.axis_index("c")
    s = lax.axis_index("s")
    wid = s * NC + c
    ch0 = wid * NCH

    isS = (isS0, isS1, isS2, isS3)
    isD = (isD0, isD1, isD2, isD3)
    isE = (isE0, isE1, isE2, isE3)
    gse = (g0, g1, g2, g3)
    sse = (s0, s1, s2, s3)
    bufs = (r0, r1, r2, r3)

    def issue_idxS(j, slot):
        pltpu.async_copy(src_hbm.at[ch0 + j], sring.at[slot], isS[slot])

    def wait_idxS(j, slot):
        pltpu.make_async_copy(src_hbm.at[ch0 + j], sring.at[slot],
                              isS[slot]).wait()

    def issue_idxD(j, slot):
        pltpu.async_copy(dst_hbm.at[ch0 + j], dring.at[slot], isD[slot])
        pltpu.async_copy(ew_hbm.at[ch0 + j], ering.at[slot], isE[slot])

    def wait_idxD(j, slot):
        pltpu.make_async_copy(dst_hbm.at[ch0 + j], dring.at[slot],
                              isD[slot]).wait()

    def wait_ew(j, slot):
        pltpu.make_async_copy(ew_hbm.at[ch0 + j], ering.at[slot],
                              isE[slot]).wait()

    def issue_gather(j, slot):
        pltpu.async_copy(h_hbm.at[sring.at[slot]], bufs[slot], gse[slot])

    def wait_gather(j, slot):
        pltpu.make_async_copy(h_hbm.at[sring.at[slot]], bufs[slot],
                              gse[slot]).wait()

    def issue_scatter(j, slot):
        pltpu.async_copy(bufs[slot], acc_sh.at[dring.at[slot]], sse[slot],
                         add=True)

    def wait_scatter(j, slot):
        pltpu.make_async_copy(bufs[slot], acc_sh.at[dring.at[slot]],
                              sse[slot]).wait()

    def scale(slot, ch):
        def sgroup(g, _):
            ewg = ering[slot, pl.ds(g * 16, 16)]
            for r2 in range(16):
                m = jnp.broadcast_to(ewg[r2], (16,))
                for f in range(D // 16):
                    sl = pl.ds(f * 16, 16)
                    bufs[slot][g * 16 + r2, sl] = bufs[slot][g * 16 + r2, sl] * m
            return 0

        lax.fori_loop(0, CB // 16, sgroup, 0)

    for k in range(3):
        issue_idxS(k, k)
    issue_idxD(0, 0)
    issue_idxD(1, 1)
    wait_idxS(0, 0)
    issue_gather(0, 0)
    wait_idxS(1, 1)
    issue_gather(1, 1)

    z = jnp.zeros((16,), jnp.float32)

    def zrow(r, _):
        for f in range(D // 16):
            r3[r, pl.ds(f * 16, 16)] = z
        return 0

    lax.fori_loop(0, CB, zrow, 0, unroll=4)
    row0 = s * RPT
    for off in range(0, RPT, CB):
        nr = min(CB, RPT - off)
        pltpu.sync_copy(r3.at[pl.ds(0, nr)], acc_sh.at[pl.ds(row0 + off, nr)])
    plsc.subcore_barrier()

    def step(j, k, *, wS, wG2, wI3):
        if wG2:
            wait_idxS(j + 2, (k + 2) % RD)
        if wS:
            wait_scatter(j - 2, (k + 2) % RD)
        if wG2:
            issue_gather(j + 2, (k + 2) % RD)
            issue_idxD(j + 2, (k + 2) % RD)
        if wI3:
            issue_idxS(j + 3, (k + 3) % RD)
        wait_gather(j, k)
        wait_ew(j, k)
        scale(k, j)
        wait_idxD(j, k)
        issue_scatter(j, k)

    step(0, 0, wS=False, wG2=True, wI3=True)
    step(1, 1, wS=False, wG2=True, wI3=True)
    step(2, 2, wS=True, wG2=True, wI3=True)
    step(3, 3, wS=True, wG2=True, wI3=True)

    def body(i, _):
        j0 = i * 4
        for k in range(4):
            step(j0 + k, k, wS=True, wG2=True, wI3=True)
        return 0

    lax.fori_loop(1, (NCH - 5) // 4, body, 0)

    step(NCH - 5, 0, wS=True, wG2=True, wI3=True)
    step(NCH - 4, 1, wS=True, wG2=True, wI3=True)
    step(NCH - 3, 2, wS=True, wG2=True, wI3=False)
    step(NCH - 2, 3, wS=True, wG2=False, wI3=False)
    step(NCH - 1, 0, wS=True, wG2=False, wI3=False)
    wait_scatter(NCH - 2, 3)
    wait_scatter(NCH - 1, 0)

    plsc.subcore_barrier()
    for off in range(0, RPT, CB):
        nr = min(CB, RPT - off)
        pltpu.sync_copy(acc_sh.at[pl.ds(row0 + off, nr)],
                        out_hbm.at[c, pl.ds(row0 + off, nr)])


def _segsum_partials(h, srcc, dstc, ewc):
    f = pl.kernel(
        _segsum_sc_body,
        out_type=jax.ShapeDtypeStruct((NC, NR, D), jnp.float32),
        mesh=_get_sc_mesh(),
        scratch_types=(
            [pltpu.VMEM((RD, CB), jnp.int32),
             pltpu.VMEM((RD, CB), jnp.int32),
             pltpu.VMEM((RD, CB), jnp.float32)]
            + [pltpu.VMEM((CB, D), jnp.float32)] * 4
            + [pltpu.VMEM_SHARED((NR, D), jnp.float32)]
            + [pltpu.SemaphoreType.DMA] * 20
        ),
    )
    return f(h, srcc, dstc, ewc)


def kernel(x, edge_attr, edge_index, W0, lin0W, W1, lin1W,
           m0_l1W, m0_l1b, m0_l2W, m0_l2b, m0_m1W, m0_m1b, m0_m2W, m0_m2b,
           m1_l1W, m1_l1b, m1_l2W, m1_l2b, m1_m1W, m1_m1b, m1_m2W, m1_m2b,
           p1W, p1b, p2W, p2b):
    srcc = edge_index[0].reshape(E // CB, CB)
    dstc = edge_index[1].reshape(E // CB, CB)
    ewc = edge_attr.reshape(E // CB, CB)

    h1 = _mm(x, m0_l1W, m0_l1b)
    ag = _segsum_partials(h1, srcc, dstc, ewc)
    xl2 = _mm(x, m0_l2W, m0_l2b)
    mask0, h2 = _mask_h2(ag, xl2, x, m0_m1W, m0_m1b, m0_m2W, m0_m2b, W0)
    ag = _segsum_partials(h2, srcc, dstc, ewc)
    x1, h1, xl2 = _comb_h1l2(ag, x, lin0W, mask0,
                             m1_l1W, m1_l1b, m1_l2W, m1_l2b)

    ag = _segsum_partials(h1, srcc, dstc, ewc)
    mask1, h2 = _mask_h2(ag, xl2, x1, m1_m1W, m1_m1b, m1_m2W, m1_m2b, W1)
    ag = _segsum_partials(h2, srcc, dstc, ewc)
    return _comb_post(ag, x1, lin1W, mask1, p1W, p1b, p2W, p2b)
